# Initial kernel scaffold; baseline (speedup 1.0000x reference)
import jax
import jax.numpy as jnp
from jax.experimental import pallas as pl


def kernel(input_ids, E, W1, b1, Wr, br, Wc, bc):
    raise NotImplementedError("write your pallas kernel here")


# speedup vs baseline: 13.0739x; 13.0739x over previous
import functools

import jax
import jax.numpy as jnp
from jax import lax
from jax.experimental import pallas as pl
from jax.experimental.pallas import tpu as pltpu
from jax.experimental.pallas import tpu_sc as plsc

B = 16384
SEQ = 200
D = 32
HID = 64

NC = 2
NS = 16
NW = NC * NS
BPW = B // NW
CB = 8
TOK = CB * SEQ
IDXW = 100
NSTREAM = TOK // IDXW
NCHUNK = BPW // CB
L = 16


def _sc_pool(ids2d, table):
  mesh = plsc.VectorSubcoreMesh(
      core_axis_name="c", subcore_axis_name="s", num_cores=NC, num_subcores=NS)

  @functools.partial(
      pl.kernel,
      out_type=jax.ShapeDtypeStruct((B, D), jnp.float32),
      mesh=mesh,
      compiler_params=pltpu.CompilerParams(use_tc_tiling_on_sc=False),
      scratch_types=[
          pltpu.VMEM((NSTREAM, IDXW), jnp.int32),
          pltpu.VMEM((TOK, D), jnp.float32),
          pltpu.VMEM((BPW, D), jnp.float32),
          pltpu.SemaphoreType.DMA,
      ],
  )
  def body(ids_hbm, tab_hbm, out_hbm, idxb, gbuf, obuf, sem_g):
    wid = lax.axis_index("s") * NC + lax.axis_index("c")
    row0 = wid * (BPW * SEQ // IDXW)

    def chunk_body(c, carry):
      r0 = row0 + c * (TOK // IDXW)
      pltpu.sync_copy(ids_hbm.at[pl.ds(r0, NSTREAM)], idxb)
      copies = [
          pltpu.make_async_copy(
              tab_hbm.at[idxb.at[r]],
              gbuf.at[pl.ds(r * IDXW, IDXW)],
              sem_g,
          )
          for r in range(NSTREAM)
      ]
      for cp in copies:
        cp.start()
      for cp in copies:
        cp.wait()

      scale = jnp.full((L,), 1.0 / SEQ, jnp.float32)
      for b in range(CB):
        base = b * SEQ

        def tok_body(t8, accs):
          a0, a1, a2, a3, a4, a5, a6, a7 = accs
          t = base + t8 * 8
          a0 = a0 + gbuf[t + 0, pl.ds(0, L)]
          a1 = a1 + gbuf[t + 0, pl.ds(L, L)]
          a2 = a2 + gbuf[t + 1, pl.ds(0, L)]
          a3 = a3 + gbuf[t + 1, pl.ds(L, L)]
          a4 = a4 + gbuf[t + 2, pl.ds(0, L)]
          a5 = a5 + gbuf[t + 2, pl.ds(L, L)]
          a6 = a6 + gbuf[t + 3, pl.ds(0, L)]
          a7 = a7 + gbuf[t + 3, pl.ds(L, L)]
          a0 = a0 + gbuf[t + 4, pl.ds(0, L)]
          a1 = a1 + gbuf[t + 4, pl.ds(L, L)]
          a2 = a2 + gbuf[t + 5, pl.ds(0, L)]
          a3 = a3 + gbuf[t + 5, pl.ds(L, L)]
          a4 = a4 + gbuf[t + 6, pl.ds(0, L)]
          a5 = a5 + gbuf[t + 6, pl.ds(L, L)]
          a6 = a6 + gbuf[t + 7, pl.ds(0, L)]
          a7 = a7 + gbuf[t + 7, pl.ds(L, L)]
          return a0, a1, a2, a3, a4, a5, a6, a7

        z = jnp.zeros((L,), jnp.float32)
        accs = lax.fori_loop(0, SEQ // 8, tok_body, (z,) * 8)
        s0 = (accs[0] + accs[2]) + (accs[4] + accs[6])
        s1 = (accs[1] + accs[3]) + (accs[5] + accs[7])
        obuf[c * CB + b, pl.ds(0, L)] = s0 * scale
        obuf[c * CB + b, pl.ds(L, L)] = s1 * scale
      return carry

    lax.fori_loop(0, NCHUNK, chunk_body, 0)
    pltpu.sync_copy(obuf, out_hbm.at[pl.ds(wid * BPW, BPW)])

  return body(ids2d, table)


def _mlp(pooled, W1, b1, Wcat, bcat):
  BM = 2048

  def body(p_ref, w1_ref, b1_ref, wc_ref, bc_ref, o_ref):
    p = p_ref[...]
    h = jnp.dot(p, w1_ref[...], preferred_element_type=jnp.float32)
    h = jnp.maximum(h + b1_ref[...], 0.0)
    o = jnp.dot(h, wc_ref[...], preferred_element_type=jnp.float32)
    o_ref[...] = o + bc_ref[...]

  return pl.pallas_call(
      body,
      grid=(B // BM,),
      in_specs=[
          pl.BlockSpec((BM, D), lambda i: (i, 0)),
          pl.BlockSpec((D, HID), lambda i: (0, 0)),
          pl.BlockSpec((1, HID), lambda i: (0, 0)),
          pl.BlockSpec((HID, 8), lambda i: (0, 0)),
          pl.BlockSpec((1, 8), lambda i: (0, 0)),
      ],
      out_specs=pl.BlockSpec((BM, 8), lambda i: (i, 0)),
      out_shape=jax.ShapeDtypeStruct((B, 8), jnp.float32),
  )(pooled, W1, b1, Wcat, bcat)


def kernel(input_ids, E, W1, b1, Wr, br, Wc, bc):
  ids2d = input_ids.astype(jnp.int32).reshape(B * SEQ // IDXW, IDXW)
  pooled = _sc_pool(ids2d, E)
  Wcat = jnp.concatenate([Wr, Wc], axis=1)
  bcat = jnp.concatenate([br, bc]).reshape(1, 8)
  out8 = _mlp(pooled, W1, b1.reshape(1, HID), Wcat, bcat)
  return out8[:, :5], out8[:, 5:]


# speedup vs baseline: 16.2009x; 1.2392x over previous
import functools

import jax
import jax.numpy as jnp
from jax import lax
from jax.experimental import pallas as pl
from jax.experimental.pallas import tpu as pltpu
from jax.experimental.pallas import tpu_sc as plsc

B = 16384
SEQ = 200
D = 32
HID = 64

NC = 2
NS = 16
NW = NC * NS
BPW = B // NW
CB = 8
TOK = CB * SEQ
IDXW = 100
NSTREAM = TOK // IDXW
NCHUNK = BPW // CB
L = 16


def _sc_pool(ids2d, table):
  mesh = plsc.VectorSubcoreMesh(
      core_axis_name="c", subcore_axis_name="s", num_cores=NC, num_subcores=NS)

  @functools.partial(
      pl.kernel,
      out_type=jax.ShapeDtypeStruct((B, D), jnp.float32),
      mesh=mesh,
      compiler_params=pltpu.CompilerParams(use_tc_tiling_on_sc=False),
      scratch_types=[
          pltpu.VMEM((2, NSTREAM, IDXW), jnp.int32),
          pltpu.VMEM((2, TOK, D), jnp.float32),
          pltpu.VMEM((BPW, D), jnp.float32),
          pltpu.SemaphoreType.DMA,
          pltpu.SemaphoreType.DMA,
          pltpu.SemaphoreType.DMA,
          pltpu.SemaphoreType.DMA,
      ],
  )
  def body(ids_hbm, tab_hbm, out_hbm, idxb, gbuf, obuf, si0, si1, sg0, sg1):
    wid = lax.axis_index("s") * NC + lax.axis_index("c")
    row0 = wid * (BPW * SEQ // IDXW)
    sem_i = (si0, si1)
    sem_g = (sg0, sg1)

    def idx_copy(c, d):
      return pltpu.make_async_copy(
          ids_hbm.at[pl.ds(row0 + c * (TOK // IDXW), NSTREAM)],
          idxb.at[d], sem_i[d])

    def gathers(d):
      return [
          pltpu.make_async_copy(
              tab_hbm.at[idxb.at[d, r]],
              gbuf.at[d, pl.ds(r * IDXW, IDXW)],
              sem_g[d],
          )
          for r in range(NSTREAM)
      ]

    cp = idx_copy(0, 0)
    cp.start()
    cp.wait()
    for g in gathers(0):
      g.start()
    idx_copy(1, 1).start()

    def pair_body(c2, carry):
      for d in range(2):
        e = 1 - d
        c = c2 * 2 + d
        @pl.when(c + 1 < NCHUNK)
        def _():
          idx_copy(c + 1, e).wait()
          for g in gathers(e):
            g.start()

        for g in gathers(d):
          g.wait()

        @pl.when(c + 2 < NCHUNK)
        def _():
          idx_copy(c + 2, d).start()

        scale = jnp.full((L,), 1.0 / SEQ, jnp.float32)
        for b in range(CB):
          base = b * SEQ

          def tok_body(t8, accs):
            a0, a1, a2, a3, a4, a5, a6, a7 = accs
            t = base + t8 * 8
            a0 = a0 + gbuf[d, t + 0, pl.ds(0, L)]
            a1 = a1 + gbuf[d, t + 0, pl.ds(L, L)]
            a2 = a2 + gbuf[d, t + 1, pl.ds(0, L)]
            a3 = a3 + gbuf[d, t + 1, pl.ds(L, L)]
            a4 = a4 + gbuf[d, t + 2, pl.ds(0, L)]
            a5 = a5 + gbuf[d, t + 2, pl.ds(L, L)]
            a6 = a6 + gbuf[d, t + 3, pl.ds(0, L)]
            a7 = a7 + gbuf[d, t + 3, pl.ds(L, L)]
            a0 = a0 + gbuf[d, t + 4, pl.ds(0, L)]
            a1 = a1 + gbuf[d, t + 4, pl.ds(L, L)]
            a2 = a2 + gbuf[d, t + 5, pl.ds(0, L)]
            a3 = a3 + gbuf[d, t + 5, pl.ds(L, L)]
            a4 = a4 + gbuf[d, t + 6, pl.ds(0, L)]
            a5 = a5 + gbuf[d, t + 6, pl.ds(L, L)]
            a6 = a6 + gbuf[d, t + 7, pl.ds(0, L)]
            a7 = a7 + gbuf[d, t + 7, pl.ds(L, L)]
            return a0, a1, a2, a3, a4, a5, a6, a7

          z = jnp.zeros((L,), jnp.float32)
          accs = lax.fori_loop(0, SEQ // 8, tok_body, (z,) * 8)
          s0 = (accs[0] + accs[2]) + (accs[4] + accs[6])
          s1 = (accs[1] + accs[3]) + (accs[5] + accs[7])
          obuf[c * CB + b, pl.ds(0, L)] = s0 * scale
          obuf[c * CB + b, pl.ds(L, L)] = s1 * scale
      return carry

    lax.fori_loop(0, NCHUNK // 2, pair_body, 0)
    pltpu.sync_copy(obuf, out_hbm.at[pl.ds(wid * BPW, BPW)])

  return body(ids2d, table)


def _mlp(pooled, W1, b1, Wcat, bcat):
  BM = 2048

  def body(p_ref, w1_ref, b1_ref, wc_ref, bc_ref, o_ref):
    p = p_ref[...]
    h = jnp.dot(p, w1_ref[...], preferred_element_type=jnp.float32)
    h = jnp.maximum(h + b1_ref[...], 0.0)
    o = jnp.dot(h, wc_ref[...], preferred_element_type=jnp.float32)
    o_ref[...] = o + bc_ref[...]

  return pl.pallas_call(
      body,
      grid=(B // BM,),
      in_specs=[
          pl.BlockSpec((BM, D), lambda i: (i, 0)),
          pl.BlockSpec((D, HID), lambda i: (0, 0)),
          pl.BlockSpec((1, HID), lambda i: (0, 0)),
          pl.BlockSpec((HID, 8), lambda i: (0, 0)),
          pl.BlockSpec((1, 8), lambda i: (0, 0)),
      ],
      out_specs=pl.BlockSpec((BM, 8), lambda i: (i, 0)),
      out_shape=jax.ShapeDtypeStruct((B, 8), jnp.float32),
  )(pooled, W1, b1, Wcat, bcat)


def kernel(input_ids, E, W1, b1, Wr, br, Wc, bc):
  ids2d = input_ids.astype(jnp.int32).reshape(B * SEQ // IDXW, IDXW)
  pooled = _sc_pool(ids2d, E)
  Wcat = jnp.concatenate([Wr, Wc], axis=1)
  bcat = jnp.concatenate([br, bc]).reshape(1, 8)
  out8 = _mlp(pooled, W1, b1.reshape(1, HID), Wcat, bcat)
  return out8[:, :5], out8[:, 5:]
